# baseline (speedup 1.0000x reference)
import functools

import jax
import jax.numpy as jnp
from jax import lax
from jax.experimental import pallas as pl
from jax.experimental.pallas import tpu as pltpu
from jax.experimental.pallas import tpu_sc as plsc

B = 4
S = 2048
EMB = 1024
SCALE = 32.0

NUM_CORES = 2
NUM_SUBCORES = 16
NUM_WORKERS = NUM_CORES * NUM_SUBCORES
ROWS = B * S
S_PER_WORKER = S // NUM_WORKERS
CHUNK = 8
NUM_CHUNKS = S_PER_WORKER // CHUNK
LANES = 16
VREGS_PER_ROW = EMB // LANES


def _make_embed_kernel():
    mesh = plsc.VectorSubcoreMesh(core_axis_name="c", subcore_axis_name="s")

    @functools.partial(
        pl.kernel,
        mesh=mesh,
        out_type=jax.ShapeDtypeStruct((ROWS, EMB), jnp.float32),
        scratch_types=[
            pltpu.VMEM((B, S_PER_WORKER), jnp.int32),
            pltpu.VMEM((B * CHUNK, EMB), jnp.float32),
            pltpu.VMEM((B * CHUNK, EMB), jnp.float32),
            pltpu.VMEM((B * CHUNK, EMB), jnp.float32),
            pltpu.VMEM((CHUNK, EMB), jnp.float32),
            pltpu.VMEM((CHUNK, EMB), jnp.float32),
            pltpu.SemaphoreType.DMA,
            pltpu.SemaphoreType.DMA,
            pltpu.SemaphoreType.DMA,
            pltpu.SemaphoreType.DMA,
            pltpu.SemaphoreType.DMA,
            pltpu.SemaphoreType.DMA,
            pltpu.SemaphoreType.DMA,
            pltpu.SemaphoreType.DMA,
        ],
    )
    def embed(x_hbm, w_hbm, pe_hbm, out_hbm,
              idx_v, wbuf0, wbuf1, wbuf2, pbuf0, pbuf1,
              gsem0, gsem1, gsem2, ssem0, ssem1, ssem2, psem0, psem1):
        wid = lax.axis_index("s") * NUM_CORES + lax.axis_index("c")
        s0 = wid * S_PER_WORKER
        wbuf = (wbuf0, wbuf1, wbuf2)
        pbuf = (pbuf0, pbuf1)
        gsem = (gsem0, gsem1, gsem2)
        ssem = (ssem0, ssem1, ssem2)
        psem = (psem0, psem1)

        xh = [
            pltpu.async_copy(
                x_hbm.at[pl.ds(b * S + s0, S_PER_WORKER)], idx_v.at[b], gsem0
            )
            for b in range(B)
        ]
        for h in xh:
            h.wait()

        def gathers_issue(c):
            return [
                pltpu.async_copy(
                    w_hbm.at[idx_v.at[b, pl.ds(c * CHUNK, CHUNK)]],
                    wbuf[c % 3].at[pl.ds(b * CHUNK, CHUNK)],
                    gsem[c % 3],
                )
                for b in range(B)
            ]

        def pe_issue(c):
            return pltpu.async_copy(
                pe_hbm.at[pl.ds(s0 + c * CHUNK, CHUNK)], pbuf[c % 2], psem[c % 2]
            )

        def stores_issue(c):
            return [
                pltpu.async_copy(
                    wbuf[c % 3].at[pl.ds(b * CHUNK, CHUNK)],
                    out_hbm.at[pl.ds(b * S + s0 + c * CHUNK, CHUNK)],
                    ssem[c % 3],
                )
                for b in range(B)
            ]

        ph = {0: pe_issue(0), 1: pe_issue(1)}
        gh = {0: gathers_issue(0), 1: gathers_issue(1)}
        sh = {}
        for c in range(NUM_CHUNKS):
            if c + 2 < NUM_CHUNKS:
                if c >= 1:
                    for h in sh[c - 1]:
                        h.wait()
                gh[c + 2] = gathers_issue(c + 2)
            ph[c].wait()
            for h in gh[c]:
                h.wait()

            wb, pb = wbuf[c % 3], pbuf[c % 2]

            @pl.loop(0, CHUNK)
            def _(r):
                for v in range(VREGS_PER_ROW):
                    sl = pl.ds(v * LANES, LANES)
                    pe = pb[r, sl]
                    for b in range(B):
                        wb[b * CHUNK + r, sl] = wb[b * CHUNK + r, sl] * SCALE + pe

            sh[c] = stores_issue(c)
            if c + 2 < NUM_CHUNKS:
                ph[c + 2] = pe_issue(c + 2)

        for c in (NUM_CHUNKS - 3, NUM_CHUNKS - 2, NUM_CHUNKS - 1):
            for h in sh[c]:
                h.wait()

    return embed


_embed = _make_embed_kernel()


def kernel(x, W, pos_emb):
    x_flat = x.reshape(ROWS).astype(jnp.int32)
    out = _embed(x_flat, W, pos_emb)
    return out.reshape(B, S, EMB)
